# baseline (speedup 1.0000x reference)
import functools

import jax
import jax.numpy as jnp
from jax import lax
from jax.experimental import pallas as pl
from jax.experimental.pallas import tpu as pltpu
from jax.experimental.pallas import tpu_sc as plsc

NC = 2
NS = 16
L = 16
NW = NC * NS

BATCH = 16384
D = 64
BLK = 128
N_U = 1000000
N_B = 100000
NBLK_U = (N_U + BLK - 1) // BLK
NBLK_B = (N_B + BLK - 1) // BLK
SLOTS_U = 246
SLOTS_B = 26
BPW = BATCH // NW
STAGE_ROWS = BATCH + 2 * L
GRP = 64

_mesh = plsc.VectorSubcoreMesh(core_axis_name="c", subcore_axis_name="s")

_STAGE = jax.ShapeDtypeStruct((STAGE_ROWS, BLK), jnp.float32)


@functools.partial(
    pl.kernel,
    out_type=(_STAGE, _STAGE),
    mesh=_mesh,
    scratch_types=[
        pltpu.VMEM((BATCH,), jnp.int32),
        pltpu.VMEM((BATCH + L,), jnp.int32),
        pltpu.VMEM((BATCH + L,), jnp.int32),
        pltpu.VMEM((BATCH + L,), jnp.int32),
        pltpu.VMEM((BATCH + L,), jnp.int32),
        pltpu.VMEM((256,), jnp.int32),
        pltpu.VMEM((256,), jnp.int32),
        pltpu.VMEM((D, BLK), jnp.float32),
        pltpu.VMEM((D, BLK), jnp.float32),
        pltpu.VMEM((GRP, BLK), jnp.float32),
        pltpu.VMEM((GRP,), jnp.int32),
        pltpu.VMEM((BATCH // L,), jnp.int32),
        pltpu.VMEM((BATCH // L + L,), jnp.int32),
        pltpu.SMEM((256,), jnp.int32),
        pltpu.SMEM((256,), jnp.int32),
        pltpu.SMEM((256,), jnp.int32),
        pltpu.SemaphoreType.DMA,
        pltpu.SemaphoreType.DMA,
        pltpu.SemaphoreType.DMA,
        pltpu.SemaphoreType.DMA,
    ],
    compiler_params=pltpu.CompilerParams(
        needs_layout_passes=False, use_tc_tiling_on_sc=True),
)
def _gather_sc(user_ids, banner_ids, ut, bt, stage_u, stage_b,
               ids_v, comp_id, comp_pos, plc_id, plc_pos,
               counts_v, starts_v, bufA, bufB, svecs, sidx_v, pcv, offv,
               smem_cur, smem_start, smem_cnt,
               semA, semB, sem_sc, sem_misc):
    wid = lax.axis_index("s") * NC + lax.axis_index("c")
    lanes = lax.iota(jnp.int32, L)
    wid_v = jnp.full((L,), 0, jnp.int32) + wid
    ones = jnp.full((L,), 1, jnp.int32)
    dump0 = jnp.full((L,), BATCH, jnp.int32)
    m0 = lanes == 0

    def side(ids_hbm, table, stage, nblk, nslots):
        pltpu.sync_copy(ids_hbm, ids_v)
        for i in range(256 // L):
            counts_v[pl.ds(i * L, L)] = jnp.zeros((L,), jnp.int32)

        @pl.loop(0, BATCH // L)
        def _cnt(i):
            v = ids_v[pl.ds(i * L, L)]
            blk = lax.shift_right_logical(v, 7)
            own = (blk & 31) == wid_v
            slot = lax.shift_right_logical(blk, 5)
            plsc.addupdate_scatter(counts_v, [slot], ones, mask=own)
            pc = plsc.all_reduce_population_count(own)
            plsc.store_scatter(pcv, [jnp.full((L,), 0, jnp.int32) + i], pc,
                               mask=m0)

        acc = jnp.int32(0)
        for t in range(BATCH // L // L):
            c = pcv[pl.ds(t * L, L)]
            cs = plsc.cumsum(c)
            offv[pl.ds(t * L, L)] = cs - c + acc
            acc = acc + cs[L - 1]

        @pl.loop(0, BATCH // L)
        def _scan(i):
            v = ids_v[pl.ds(i * L, L)]
            pos = lanes + i * L
            blk = lax.shift_right_logical(v, 7)
            own = (blk & 31) == wid_v
            off = offv[pl.ds(i, L)][0]
            plsc.store_compressed(comp_id.at[pl.ds(off, L)], v, mask=own)
            plsc.store_compressed(comp_pos.at[pl.ds(off, L)], pos, mask=own)

        carry = jnp.int32(0)
        for i in range(256 // L):
            c = counts_v[pl.ds(i * L, L)]
            cs = plsc.cumsum(c)
            starts_v[pl.ds(i * L, L)] = cs - c + carry
            carry = carry + cs[L - 1]
        for i in range(256 // L):
            s = starts_v[pl.ds(i * L, L)]
            c = counts_v[pl.ds(i * L, L)]
            for j in range(L):
                smem_cur[i * L + j] = s[j]
                smem_start[i * L + j] = s[j]
                smem_cnt[i * L + j] = c[j]

        n = carry

        @pl.loop(0, n)
        def _place(i):
            c = comp_id[pl.ds(i, L)][0]
            p = comp_pos[pl.ds(i, L)][0]
            slot = lax.shift_right_logical(c, 12)
            cur = smem_cur[slot]
            smem_cur[slot] = cur + 1
            m0 = lanes == 0
            cv = jnp.full((L,), 0, jnp.int32) + cur
            plsc.store_scatter(plc_id, [cv], jnp.full((L,), 0, jnp.int32) + c, mask=m0)
            plsc.store_scatter(plc_pos, [cv], jnp.full((L,), 0, jnp.int32) + p, mask=m0)

        def reset_sidx():
            for k in range(GRP // L):
                sidx_v[pl.ds(k * L, L)] = dump0 + k * L + lanes

        reset_sidx()

        def fire(s, buf, sem):
            @pl.when(smem_cnt[s] > 0)
            def _():
                b = jnp.minimum(s * 32 + wid, nblk - 1)
                col = pl.multiple_of(b * BLK, BLK)
                pltpu.async_copy(table.at[:, pl.ds(col, BLK)], buf, sem)

        def drain(s, buf, sem):
            @pl.when(smem_cnt[s] > 0)
            def _():
                pltpu.make_async_copy(
                    table.at[:, pl.ds(0, BLK)], buf, sem).wait()

        def flush():
            pltpu.async_copy(svecs, stage.at[sidx_v], sem_sc).wait()
            reset_sidx()

        def process(s, buf):
            st = smem_start[s]
            cnt = smem_cnt[s]

            @pl.loop(st, st + cnt)
            def _entry(i):
                c = plc_id[pl.ds(i, L)][0]
                p = plc_pos[pl.ds(i, L)][0]
                l = c & (BLK - 1)
                lv = jnp.full((L,), 0, jnp.int32) + l
                r = i & (GRP - 1)
                for k in range(D // L):
                    g = plsc.load_gather(bufA if buf is None else buf,
                                         [lanes + k * L, lv])
                    svecs[r, pl.ds(k * L, L)] = g
                plsc.store_scatter(sidx_v,
                                   [jnp.full((L,), 0, jnp.int32) + r],
                                   jnp.full((L,), 0, jnp.int32) + p,
                                   mask=lanes == 0)

                @pl.when(r == GRP - 1)
                def _():
                    flush()

        fire(0, bufA, semA)

        @pl.loop(0, nslots // 2)
        def _pair(t):
            s0 = 2 * t
            fire(s0 + 1, bufB, semB)
            drain(s0, bufA, semA)
            process(s0, bufA)

            @pl.when(t < nslots // 2 - 1)
            def _():
                fire(s0 + 2, bufA, semA)

            drain(s0 + 1, bufB, semB)
            process(s0 + 1, bufB)

        flush()

    side(user_ids, ut, stage_u, NBLK_U, SLOTS_U)
    side(banner_ids, bt, stage_b, NBLK_B, SLOTS_B)


@functools.partial(
    pl.kernel,
    out_type=jax.ShapeDtypeStruct((BATCH,), jnp.float32),
    mesh=_mesh,
    scratch_types=[
        pltpu.VMEM((BPW, D), jnp.float32),
        pltpu.VMEM((BPW, D), jnp.float32),
        pltpu.VMEM((BPW,), jnp.float32),
        pltpu.SemaphoreType.DMA,
        pltpu.SemaphoreType.DMA,
    ],
    compiler_params=pltpu.CompilerParams(
        needs_layout_passes=False, use_tc_tiling_on_sc=False),
)
def _dot_sc(stage_u, stage_b, out_hbm, uv, bv, out_v, sem_u, sem_b):
    wid = lax.axis_index("s") * NC + lax.axis_index("c")
    base = wid * BPW
    cu = pltpu.async_copy(
        stage_u.at[pl.ds(base, BPW), pl.ds(0, D)], uv, sem_u)
    cb = pltpu.async_copy(
        stage_b.at[pl.ds(base, BPW), pl.ds(0, D)], bv, sem_b)
    cu.wait()
    cb.wait()

    lanes = lax.iota(jnp.int32, L)

    @pl.loop(0, BPW // L)
    def _group(g):
        acc = jnp.zeros((L,), jnp.float32)
        for j in range(L):
            e = g * L + j
            ss = jnp.zeros((L,), jnp.float32)
            for c in range(D // L):
                ss = ss + uv[e, pl.ds(c * L, L)] * bv[e, pl.ds(c * L, L)]
            acc = jnp.where(lanes == j, jnp.sum(ss), acc)
        out_v[pl.ds(g * L, L)] = acc

    pltpu.sync_copy(out_v, out_hbm.at[pl.ds(base, BPW)])


def kernel(user_ids, banner_ids, user_table, banner_table):
    stage_u, stage_b = _gather_sc(
        user_ids, banner_ids, user_table.T, banner_table.T)
    return _dot_sc(stage_u, stage_b)


# speedup vs baseline: 1.1018x; 1.1018x over previous
import functools

import jax
import jax.numpy as jnp
from jax import lax
from jax.experimental import pallas as pl
from jax.experimental.pallas import tpu as pltpu
from jax.experimental.pallas import tpu_sc as plsc

NC = 2
NS = 16
L = 16
NW = NC * NS

BATCH = 16384
D = 64
BLK = 128
N_U = 1000000
N_B = 100000
NBLK_U = (N_U + BLK - 1) // BLK
NBLK_B = (N_B + BLK - 1) // BLK
SLOTS_U = 246
SLOTS_B = 26
BPW = BATCH // NW
STAGE_ROWS = BATCH + 2 * L
GRP = 64

_mesh = plsc.VectorSubcoreMesh(core_axis_name="c", subcore_axis_name="s")

_STAGE = jax.ShapeDtypeStruct((STAGE_ROWS, BLK), jnp.float32)


@functools.partial(
    pl.kernel,
    out_type=(_STAGE, _STAGE),
    mesh=_mesh,
    scratch_types=[
        pltpu.VMEM((BATCH,), jnp.int32),
        pltpu.VMEM((BATCH + L,), jnp.int32),
        pltpu.VMEM((BATCH + L,), jnp.int32),
        pltpu.VMEM((BATCH + L,), jnp.int32),
        pltpu.VMEM((BATCH + L,), jnp.int32),
        pltpu.VMEM((256,), jnp.int32),
        pltpu.VMEM((256,), jnp.int32),
        pltpu.VMEM((D, BLK), jnp.float32),
        pltpu.VMEM((D, BLK), jnp.float32),
        pltpu.VMEM((GRP, BLK), jnp.float32),
        pltpu.VMEM((GRP,), jnp.int32),
        pltpu.VMEM((BATCH // L,), jnp.int32),
        pltpu.VMEM((BATCH // L + L,), jnp.int32),
        pltpu.SMEM((256,), jnp.int32),
        pltpu.SMEM((256,), jnp.int32),
        pltpu.SMEM((256,), jnp.int32),
        pltpu.SemaphoreType.DMA,
        pltpu.SemaphoreType.DMA,
        pltpu.SemaphoreType.DMA,
        pltpu.SemaphoreType.DMA,
    ],
    compiler_params=pltpu.CompilerParams(
        needs_layout_passes=False, use_tc_tiling_on_sc=True),
)
def _gather_sc(user_ids, banner_ids, ut, bt, stage_u, stage_b,
               ids_v, comp_id, comp_pos, plc_id, plc_pos,
               counts_v, starts_v, bufA, bufB, svecs, sidx_v, pcv, offv,
               smem_cur, smem_start, smem_cnt,
               semA, semB, sem_sc, sem_misc):
    wid = lax.axis_index("s") * NC + lax.axis_index("c")
    lanes = lax.iota(jnp.int32, L)
    wid_v = jnp.full((L,), 0, jnp.int32) + wid
    ones = jnp.full((L,), 1, jnp.int32)
    dump0 = jnp.full((L,), BATCH, jnp.int32)
    m0 = lanes == 0

    def side(ids_hbm, table, stage, nblk, nslots):
        pltpu.sync_copy(ids_hbm, ids_v)
        for i in range(256 // L):
            counts_v[pl.ds(i * L, L)] = jnp.zeros((L,), jnp.int32)

        @pl.loop(0, BATCH // L, init_carry=jnp.int32(0))
        def _scan(i, off):
            v = ids_v[pl.ds(i * L, L)]
            pos = lanes + i * L
            blk = lax.shift_right_logical(v, 7)
            own = (blk & 31) == wid_v
            slot = lax.shift_right_logical(blk, 5)
            plsc.store_compressed(comp_id.at[pl.ds(off, L)], v, mask=own)
            plsc.store_compressed(comp_pos.at[pl.ds(off, L)], pos, mask=own)
            plsc.addupdate_scatter(counts_v, [slot], ones, mask=own)
            pc = plsc.all_reduce_population_count(own)
            return off + pc[0]

        carry = jnp.int32(0)
        for i in range(256 // L):
            c = counts_v[pl.ds(i * L, L)]
            cs = plsc.cumsum(c)
            starts_v[pl.ds(i * L, L)] = cs - c + carry
            carry = carry + cs[L - 1]
        for i in range(256 // L):
            s = starts_v[pl.ds(i * L, L)]
            c = counts_v[pl.ds(i * L, L)]
            for j in range(L):
                smem_cur[i * L + j] = s[j]
                smem_start[i * L + j] = s[j]
                smem_cnt[i * L + j] = c[j]

        n = carry

        @pl.loop(0, n)
        def _place(i):
            c = comp_id[pl.ds(i, L)][0]
            p = comp_pos[pl.ds(i, L)][0]
            slot = lax.shift_right_logical(c, 12)
            cur = smem_cur[slot]
            smem_cur[slot] = cur + 1
            m0 = lanes == 0
            cv = jnp.full((L,), 0, jnp.int32) + cur
            plsc.store_scatter(plc_id, [cv], jnp.full((L,), 0, jnp.int32) + c, mask=m0)
            plsc.store_scatter(plc_pos, [cv], jnp.full((L,), 0, jnp.int32) + p, mask=m0)

        def reset_sidx():
            for k in range(GRP // L):
                sidx_v[pl.ds(k * L, L)] = dump0 + k * L + lanes

        reset_sidx()

        def fire(s, buf, sem):
            @pl.when(smem_cnt[s] > 0)
            def _():
                b = jnp.minimum(s * 32 + wid, nblk - 1)
                col = pl.multiple_of(b * BLK, BLK)
                pltpu.async_copy(table.at[:, pl.ds(col, BLK)], buf, sem)

        def drain(s, buf, sem):
            @pl.when(smem_cnt[s] > 0)
            def _():
                pltpu.make_async_copy(
                    table.at[:, pl.ds(0, BLK)], buf, sem).wait()

        def flush():
            pltpu.async_copy(svecs, stage.at[sidx_v], sem_sc).wait()
            reset_sidx()

        def process(s, buf):
            st = smem_start[s]
            cnt = smem_cnt[s]

            @pl.loop(st, st + cnt)
            def _entry(i):
                c = plc_id[pl.ds(i, L)][0]
                p = plc_pos[pl.ds(i, L)][0]
                l = c & (BLK - 1)
                lv = jnp.full((L,), 0, jnp.int32) + l
                r = i & (GRP - 1)
                for k in range(D // L):
                    g = plsc.load_gather(bufA if buf is None else buf,
                                         [lanes + k * L, lv])
                    svecs[r, pl.ds(k * L, L)] = g
                plsc.store_scatter(sidx_v,
                                   [jnp.full((L,), 0, jnp.int32) + r],
                                   jnp.full((L,), 0, jnp.int32) + p,
                                   mask=lanes == 0)

                @pl.when(r == GRP - 1)
                def _():
                    flush()

        fire(0, bufA, semA)

        @pl.loop(0, nslots // 2)
        def _pair(t):
            s0 = 2 * t
            fire(s0 + 1, bufB, semB)
            drain(s0, bufA, semA)
            process(s0, bufA)

            @pl.when(t < nslots // 2 - 1)
            def _():
                fire(s0 + 2, bufA, semA)

            drain(s0 + 1, bufB, semB)
            process(s0 + 1, bufB)

        flush()

    side(user_ids, ut, stage_u, NBLK_U, SLOTS_U)
    side(banner_ids, bt, stage_b, NBLK_B, SLOTS_B)


@functools.partial(
    pl.kernel,
    out_type=jax.ShapeDtypeStruct((BATCH,), jnp.float32),
    mesh=_mesh,
    scratch_types=[
        pltpu.VMEM((BPW, D), jnp.float32),
        pltpu.VMEM((BPW, D), jnp.float32),
        pltpu.VMEM((BPW,), jnp.float32),
        pltpu.SemaphoreType.DMA,
        pltpu.SemaphoreType.DMA,
    ],
    compiler_params=pltpu.CompilerParams(
        needs_layout_passes=False, use_tc_tiling_on_sc=False),
)
def _dot_sc(stage_u, stage_b, out_hbm, uv, bv, out_v, sem_u, sem_b):
    wid = lax.axis_index("s") * NC + lax.axis_index("c")
    base = wid * BPW
    cu = pltpu.async_copy(
        stage_u.at[pl.ds(base, BPW), pl.ds(0, D)], uv, sem_u)
    cb = pltpu.async_copy(
        stage_b.at[pl.ds(base, BPW), pl.ds(0, D)], bv, sem_b)
    cu.wait()
    cb.wait()

    lanes = lax.iota(jnp.int32, L)

    @pl.loop(0, BPW // L)
    def _group(g):
        acc = jnp.zeros((L,), jnp.float32)
        for j in range(L):
            e = g * L + j
            ss = jnp.zeros((L,), jnp.float32)
            for c in range(D // L):
                ss = ss + uv[e, pl.ds(c * L, L)] * bv[e, pl.ds(c * L, L)]
            acc = jnp.where(lanes == j, jnp.sum(ss), acc)
        out_v[pl.ds(g * L, L)] = acc

    pltpu.sync_copy(out_v, out_hbm.at[pl.ds(base, BPW)])


def kernel(user_ids, banner_ids, user_table, banner_table):
    stage_u, stage_b = _gather_sc(
        user_ids, banner_ids, user_table.T, banner_table.T)
    return _dot_sc(stage_u, stage_b)


# speedup vs baseline: 1.1162x; 1.0131x over previous
import functools

import jax
import jax.numpy as jnp
from jax import lax
from jax.experimental import pallas as pl
from jax.experimental.pallas import tpu as pltpu
from jax.experimental.pallas import tpu_sc as plsc

NC = 2
NS = 16
L = 16
NW = NC * NS

BATCH = 16384
D = 64
BLK = 128
N_U = 1000000
N_B = 100000
NBLK_U = (N_U + BLK - 1) // BLK
NBLK_B = (N_B + BLK - 1) // BLK
SLOTS_U = 246
SLOTS_B = 26
BPW = BATCH // NW
STAGE_ROWS = BATCH + 2 * L
GRP = 64

_mesh = plsc.VectorSubcoreMesh(core_axis_name="c", subcore_axis_name="s")

_STAGE = jax.ShapeDtypeStruct((STAGE_ROWS, BLK), jnp.float32)


@functools.partial(
    pl.kernel,
    out_type=(_STAGE, _STAGE),
    mesh=_mesh,
    scratch_types=[
        pltpu.VMEM((BATCH,), jnp.int32),
        pltpu.VMEM((BATCH + L,), jnp.int32),
        pltpu.VMEM((BATCH + L,), jnp.int32),
        pltpu.VMEM((BATCH + L,), jnp.int32),
        pltpu.VMEM((BATCH + L,), jnp.int32),
        pltpu.VMEM((256,), jnp.int32),
        pltpu.VMEM((256,), jnp.int32),
        pltpu.VMEM((D, BLK), jnp.float32),
        pltpu.VMEM((D, BLK), jnp.float32),
        pltpu.VMEM((GRP, BLK), jnp.float32),
        pltpu.VMEM((GRP,), jnp.int32),
        pltpu.VMEM((BATCH // L,), jnp.int32),
        pltpu.VMEM((BATCH // L + L,), jnp.int32),
        pltpu.SMEM((256,), jnp.int32),
        pltpu.SMEM((256,), jnp.int32),
        pltpu.SMEM((256,), jnp.int32),
        pltpu.SemaphoreType.DMA,
        pltpu.SemaphoreType.DMA,
        pltpu.SemaphoreType.DMA,
        pltpu.SemaphoreType.DMA,
    ],
    compiler_params=pltpu.CompilerParams(
        needs_layout_passes=False, use_tc_tiling_on_sc=True),
)
def _gather_sc(user_ids, banner_ids, ut, bt, stage_u, stage_b,
               ids_v, comp_id, comp_pos, plc_id, plc_pos,
               counts_v, starts_v, bufA, bufB, svecs, sidx_v, pcv, offv,
               smem_cur, smem_start, smem_cnt,
               semA, semB, sem_sc, sem_misc):
    wid = lax.axis_index("s") * NC + lax.axis_index("c")
    lanes = lax.iota(jnp.int32, L)
    wid_v = jnp.full((L,), 0, jnp.int32) + wid
    ones = jnp.full((L,), 1, jnp.int32)
    dump0 = jnp.full((L,), BATCH, jnp.int32)
    m0 = lanes == 0

    def side(ids_hbm, table, stage, nblk, nslots):
        pltpu.sync_copy(ids_hbm, ids_v)
        for i in range(256 // L):
            counts_v[pl.ds(i * L, L)] = jnp.zeros((L,), jnp.int32)

        @pl.loop(0, BATCH // L, init_carry=jnp.int32(0), unroll=4)
        def _scan(i, off):
            v = ids_v[pl.ds(i * L, L)]
            pos = lanes + i * L
            blk = lax.shift_right_logical(v, 7)
            own = (blk & 31) == wid_v
            slot = lax.shift_right_logical(blk, 5)
            plsc.store_compressed(comp_id.at[pl.ds(off, L)], v, mask=own)
            plsc.store_compressed(comp_pos.at[pl.ds(off, L)], pos, mask=own)
            plsc.addupdate_scatter(counts_v, [slot], ones, mask=own)
            pc = plsc.all_reduce_population_count(own)
            return off + pc[0]

        carry = jnp.int32(0)
        for i in range(256 // L):
            c = counts_v[pl.ds(i * L, L)]
            cs = plsc.cumsum(c)
            starts_v[pl.ds(i * L, L)] = cs - c + carry
            carry = carry + cs[L - 1]
        for i in range(256 // L):
            s = starts_v[pl.ds(i * L, L)]
            c = counts_v[pl.ds(i * L, L)]
            for j in range(L):
                smem_cur[i * L + j] = s[j]
                smem_start[i * L + j] = s[j]
                smem_cnt[i * L + j] = c[j]

        n = carry

        @pl.loop(0, n)
        def _place(i):
            c = comp_id[pl.ds(i, L)][0]
            p = comp_pos[pl.ds(i, L)][0]
            slot = lax.shift_right_logical(c, 12)
            cur = smem_cur[slot]
            smem_cur[slot] = cur + 1
            m0 = lanes == 0
            cv = jnp.full((L,), 0, jnp.int32) + cur
            plsc.store_scatter(plc_id, [cv], jnp.full((L,), 0, jnp.int32) + c, mask=m0)
            plsc.store_scatter(plc_pos, [cv], jnp.full((L,), 0, jnp.int32) + p, mask=m0)

        def reset_sidx():
            for k in range(GRP // L):
                sidx_v[pl.ds(k * L, L)] = dump0 + k * L + lanes

        reset_sidx()

        def fire(s, buf, sem):
            @pl.when(smem_cnt[s] > 0)
            def _():
                b = jnp.minimum(s * 32 + wid, nblk - 1)
                col = pl.multiple_of(b * BLK, BLK)
                pltpu.async_copy(table.at[:, pl.ds(col, BLK)], buf, sem)

        def drain(s, buf, sem):
            @pl.when(smem_cnt[s] > 0)
            def _():
                pltpu.make_async_copy(
                    table.at[:, pl.ds(0, BLK)], buf, sem).wait()

        def flush():
            pltpu.async_copy(svecs, stage.at[sidx_v], sem_sc).wait()
            reset_sidx()

        def process(s, buf):
            st = smem_start[s]
            cnt = smem_cnt[s]

            @pl.loop(st, st + cnt)
            def _entry(i):
                c = plc_id[pl.ds(i, L)][0]
                p = plc_pos[pl.ds(i, L)][0]
                l = c & (BLK - 1)
                lv = jnp.full((L,), 0, jnp.int32) + l
                r = i & (GRP - 1)
                for k in range(D // L):
                    g = plsc.load_gather(bufA if buf is None else buf,
                                         [lanes + k * L, lv])
                    svecs[r, pl.ds(k * L, L)] = g
                plsc.store_scatter(sidx_v,
                                   [jnp.full((L,), 0, jnp.int32) + r],
                                   jnp.full((L,), 0, jnp.int32) + p,
                                   mask=lanes == 0)

                @pl.when(r == GRP - 1)
                def _():
                    flush()

        fire(0, bufA, semA)

        @pl.loop(0, nslots // 2)
        def _pair(t):
            s0 = 2 * t
            fire(s0 + 1, bufB, semB)
            drain(s0, bufA, semA)
            process(s0, bufA)

            @pl.when(t < nslots // 2 - 1)
            def _():
                fire(s0 + 2, bufA, semA)

            drain(s0 + 1, bufB, semB)
            process(s0 + 1, bufB)

        flush()

    side(user_ids, ut, stage_u, NBLK_U, SLOTS_U)
    side(banner_ids, bt, stage_b, NBLK_B, SLOTS_B)


@functools.partial(
    pl.kernel,
    out_type=jax.ShapeDtypeStruct((BATCH,), jnp.float32),
    mesh=_mesh,
    scratch_types=[
        pltpu.VMEM((BPW, D), jnp.float32),
        pltpu.VMEM((BPW, D), jnp.float32),
        pltpu.VMEM((BPW,), jnp.float32),
        pltpu.SemaphoreType.DMA,
        pltpu.SemaphoreType.DMA,
    ],
    compiler_params=pltpu.CompilerParams(
        needs_layout_passes=False, use_tc_tiling_on_sc=False),
)
def _dot_sc(stage_u, stage_b, out_hbm, uv, bv, out_v, sem_u, sem_b):
    wid = lax.axis_index("s") * NC + lax.axis_index("c")
    base = wid * BPW
    cu = pltpu.async_copy(
        stage_u.at[pl.ds(base, BPW), pl.ds(0, D)], uv, sem_u)
    cb = pltpu.async_copy(
        stage_b.at[pl.ds(base, BPW), pl.ds(0, D)], bv, sem_b)
    cu.wait()
    cb.wait()

    lanes = lax.iota(jnp.int32, L)

    @pl.loop(0, BPW // L)
    def _group(g):
        acc = jnp.zeros((L,), jnp.float32)
        for j in range(L):
            e = g * L + j
            ss = jnp.zeros((L,), jnp.float32)
            for c in range(D // L):
                ss = ss + uv[e, pl.ds(c * L, L)] * bv[e, pl.ds(c * L, L)]
            acc = jnp.where(lanes == j, jnp.sum(ss), acc)
        out_v[pl.ds(g * L, L)] = acc

    pltpu.sync_copy(out_v, out_hbm.at[pl.ds(base, BPW)])


def kernel(user_ids, banner_ids, user_table, banner_table):
    stage_u, stage_b = _gather_sc(
        user_ids, banner_ids, user_table.T, banner_table.T)
    return _dot_sc(stage_u, stage_b)
